# Initial kernel scaffold; baseline (speedup 1.0000x reference)
import jax
import jax.numpy as jnp
from jax.experimental import pallas as pl


def kernel(x, edge_index, batch, params):
    raise NotImplementedError("write your pallas kernel here")


# speedup vs baseline: 32.1554x; 32.1554x over previous
import functools

import jax
import jax.numpy as jnp
from jax import lax
from jax.experimental import pallas as pl
from jax.experimental.pallas import tpu as pltpu
from jax.experimental.pallas import tpu_sc as plsc

N = 10000
D = 128
H = 8
DH = 16
G = 16
BLK = 128
NP = 10112
NB = NP // BLK


def _prep_body(y_ref, a_ref, c_ref, w_ref, am_ref, h_ref, asd_ref):
    i = pl.program_id(0)
    y = y_ref[...]
    xin = jnp.dot(y, a_ref[...], preferred_element_type=jnp.float32) + c_ref[...]
    xin = jnp.maximum(xin, 0.0)
    rid = i * BLK + lax.broadcasted_iota(jnp.int32, (BLK, D), 0)
    xin = jnp.where(rid < N, xin, 0.0)
    h = jnp.dot(xin, w_ref[...], preferred_element_type=jnp.float32)
    h_ref[...] = h
    asd_ref[...] = jnp.dot(h, am_ref[...], preferred_element_type=jnp.float32)


def _prep(y, a, c, w, am):
    return pl.pallas_call(
        _prep_body,
        grid=(NB,),
        in_specs=[
            pl.BlockSpec((BLK, D), lambda i: (i, 0)),
            pl.BlockSpec((D, D), lambda i: (0, 0)),
            pl.BlockSpec((1, D), lambda i: (0, 0)),
            pl.BlockSpec((D, D), lambda i: (0, 0)),
            pl.BlockSpec((D, D), lambda i: (0, 0)),
        ],
        out_specs=[
            pl.BlockSpec((BLK, D), lambda i: (i, 0)),
            pl.BlockSpec((BLK, D), lambda i: (i, 0)),
        ],
        out_shape=[
            jax.ShapeDtypeStruct((NP, D), jnp.float32),
            jax.ShapeDtypeStruct((NP, D), jnp.float32),
        ],
    )(y, a, c, w, am)


def _post_body(z_ref, m_ref, s_ref, bias_ref, y_ref, ssum_ref, ssq_ref):
    i = pl.program_id(0)
    z = z_ref[0] + z_ref[1]
    zinv = 1.0 / (z + 1e-16)
    zb = jnp.dot(zinv, s_ref[...], preferred_element_type=jnp.float32)
    y = (m_ref[0] + m_ref[1]) * zb + bias_ref[...]
    rid = i * BLK + lax.broadcasted_iota(jnp.int32, (BLK, D), 0)
    y = jnp.where(rid < N, y, 0.0)
    y_ref[...] = y

    @pl.when(i == 0)
    def _():
        ssum_ref[...] = jnp.zeros_like(ssum_ref)
        ssq_ref[...] = jnp.zeros_like(ssq_ref)

    s = jnp.sum(y, axis=0)
    sq = jnp.sum(y * y, axis=0)
    ssum_ref[...] += jnp.broadcast_to(s, (8, D))
    ssq_ref[...] += jnp.broadcast_to(sq, (8, D))


def _post(z2, m2, smat, bias):
    return pl.pallas_call(
        _post_body,
        grid=(NB,),
        in_specs=[
            pl.BlockSpec((2, BLK, 16), lambda i: (0, i, 0)),
            pl.BlockSpec((2, BLK, D), lambda i: (0, i, 0)),
            pl.BlockSpec((16, D), lambda i: (0, 0)),
            pl.BlockSpec((1, D), lambda i: (0, 0)),
        ],
        out_specs=[
            pl.BlockSpec((BLK, D), lambda i: (i, 0)),
            pl.BlockSpec((8, D), lambda i: (0, 0)),
            pl.BlockSpec((8, D), lambda i: (0, 0)),
        ],
        out_shape=[
            jax.ShapeDtypeStruct((NP, D), jnp.float32),
            jax.ShapeDtypeStruct((8, D), jnp.float32),
            jax.ShapeDtypeStruct((8, D), jnp.float32),
        ],
    )(z2, m2, smat, bias)


def _var_body(y_ref, mu_ref, sv_ref):
    i = pl.program_id(0)
    d = y_ref[...] - mu_ref[...]
    rid = i * BLK + lax.broadcasted_iota(jnp.int32, (BLK, D), 0)
    d = jnp.where(rid < N, d, 0.0)

    @pl.when(i == 0)
    def _():
        sv_ref[...] = jnp.zeros_like(sv_ref)

    sv_ref[...] += jnp.broadcast_to(jnp.sum(d * d, axis=0), (8, D))


def _var(y, mu_row):
    return pl.pallas_call(
        _var_body,
        grid=(NB,),
        in_specs=[
            pl.BlockSpec((BLK, D), lambda i: (i, 0)),
            pl.BlockSpec((1, D), lambda i: (0, 0)),
        ],
        out_specs=pl.BlockSpec((8, D), lambda i: (0, 0)),
        out_shape=jax.ShapeDtypeStruct((8, D), jnp.float32),
    )(y, mu_row)


def _pool_body(y_ref, g_ref, b_ref, bf_ref, s_ref, c_ref):
    i = pl.program_id(0)
    x = jnp.maximum(y_ref[...] * g_ref[...] + b_ref[...], 0.0)
    rid = i * BLK + lax.broadcasted_iota(jnp.int32, (BLK, D), 0)
    x = jnp.where(rid < N, x, 0.0)
    seg = lax.broadcasted_iota(jnp.int32, (BLK, G), 1)
    oh = (bf_ref[...] == seg).astype(jnp.float32)

    @pl.when(i == 0)
    def _():
        s_ref[...] = jnp.zeros_like(s_ref)
        c_ref[...] = jnp.zeros_like(c_ref)

    s_ref[...] += lax.dot_general(oh, x, (((0,), (0,)), ((), ())),
                                  preferred_element_type=jnp.float32)
    c_ref[...] += jnp.broadcast_to(jnp.sum(oh, axis=0)[:, None], (G, D))


def _pool(y, g, b, batchf):
    return pl.pallas_call(
        _pool_body,
        grid=(NB,),
        in_specs=[
            pl.BlockSpec((BLK, D), lambda i: (i, 0)),
            pl.BlockSpec((1, D), lambda i: (0, 0)),
            pl.BlockSpec((1, D), lambda i: (0, 0)),
            pl.BlockSpec((BLK, G), lambda i: (i, 0)),
        ],
        out_specs=[
            pl.BlockSpec((G, D), lambda i: (0, 0)),
            pl.BlockSpec((G, D), lambda i: (0, 0)),
        ],
        out_shape=[
            jax.ShapeDtypeStruct((G, D), jnp.float32),
            jax.ShapeDtypeStruct((G, D), jnp.float32),
        ],
    )(y, g, b, batchf)


def _head_body(s_ref, c_ref, w1_ref, b1_ref, w2_ref, b2_ref, o_ref):
    pooled = s_ref[...] / jnp.maximum(c_ref[...], 1.0)
    hdd = jnp.maximum(
        jnp.dot(pooled, w1_ref[...], preferred_element_type=jnp.float32)
        + b1_ref[...], 0.0)
    o_ref[...] = jnp.dot(hdd, w2_ref[...],
                         preferred_element_type=jnp.float32) + b2_ref[...]


def _head(s, c, w1p, b1p, w2p, b2p):
    return pl.pallas_call(
        _head_body,
        grid=(1,),
        in_specs=[pl.BlockSpec((G, D), lambda i: (0, 0)),
                  pl.BlockSpec((G, D), lambda i: (0, 0)),
                  pl.BlockSpec((D, D), lambda i: (0, 0)),
                  pl.BlockSpec((1, D), lambda i: (0, 0)),
                  pl.BlockSpec((D, D), lambda i: (0, 0)),
                  pl.BlockSpec((1, D), lambda i: (0, 0))],
        out_specs=pl.BlockSpec((G, D), lambda i: (0, 0)),
        out_shape=jax.ShapeDtypeStruct((G, D), jnp.float32),
    )(s, c, w1p, b1p, w2p, b2p)


NC = 2
NSC = 16
NW = NC * NSC
C = 64
EP = ((320000 + N + C * NW - 1) // (C * NW)) * (C * NW)
TPT = EP // NW
CH = TPT // C
RPT = NP // NSC
NPB = NP // 8
NPZ = 1280
ZRT = NPZ // NSC


def _edge_sc_body(sp_hbm, dp_hbm, ast_hbm, h_hbm, shift_hbm, zero_hbm,
                  z_out, m_out,
                  si, di, ib, ra, rb, wb, hb, shiftv,
                  z_sh, o_sh, sem1, sem2, sem3):
    cid = lax.axis_index("c")
    sid = lax.axis_index("s")
    wid = sid * NC + cid

    r0 = sid * RPT
    z0 = sid * ZRT
    pltpu.sync_copy(zero_hbm, o_sh.at[pl.ds(r0, RPT)])
    pltpu.sync_copy(zero_hbm.at[pl.ds(0, ZRT)], z_sh.at[pl.ds(z0, ZRT)])
    pltpu.sync_copy(shift_hbm, shiftv)
    plsc.subcore_barrier()

    shv = shiftv[...]
    zeros16 = jnp.zeros((16,), jnp.float32)
    lane = lax.iota(jnp.int32, 16)
    lane7 = lane & 7
    perm_hi = lane7 + 8
    lane_lo = lane < 8

    def chunk(g, carry):
        base = wid * TPT + g * C
        cp1 = pltpu.async_copy(sp_hbm.at[pl.ds(base, C)], si, sem1)
        cp2 = pltpu.async_copy(dp_hbm.at[pl.ds(base, C)], di, sem2)
        cp1.wait()
        cp2.wait()
        ga = pltpu.async_copy(ast_hbm.at[si], ra, sem1)
        gb = pltpu.async_copy(ast_hbm.at[di], rb, sem2)
        gh = pltpu.async_copy(h_hbm.at[si], hb, sem3)

        def mkib(k, c2):
            sl = pl.ds(k * 16, 16)
            ib[sl] = lax.shift_right_logical(di[sl], 3)
            return c2

        lax.fori_loop(0, C // 16, mkib, 0)
        ga.wait()
        gb.wait()
        gh.wait()

        def grp(j, c2):
            for r2 in range(16):
                r = j * 16 + r2
                rsp = lane * 0 + r
                e = ra[r, pl.ds(0, 16)] + plsc.load_gather(rb, [rsp, perm_hi])
                el = jnp.where(e >= 0.0, e, 0.2 * e)
                w = jnp.where(lane_lo, jnp.exp(el - shv), 0.0)
                dspl = plsc.load_gather(di, [rsp])
                osp = (dspl & 7) * 16
                for s in range(8):
                    wb[r, pl.ds(s * 16, 16)] = zeros16
                plsc.store_scatter(wb, [rsp, osp + lane], w)
                for hh in range(H):
                    spl = plsc.load_gather(wb, [rsp, osp + hh])
                    sl = pl.ds(hh * 16, 16)
                    hb[r, sl] = hb[r, sl] * spl
            return c2

        lax.fori_loop(0, C // 16, grp, 0)
        pltpu.sync_copy(wb, z_sh.at[ib], add=True)
        pltpu.sync_copy(hb, o_sh.at[di], add=True)
        return carry

    lax.fori_loop(0, CH, chunk, 0)
    plsc.subcore_barrier()
    pltpu.sync_copy(z_sh.at[pl.ds(z0, ZRT)], z_out.at[cid, pl.ds(z0, ZRT)])
    pltpu.sync_copy(o_sh.at[pl.ds(r0, RPT)], m_out.at[cid, pl.ds(r0, RPT)])


def _edge_sc(sp, dp, ast, h, shift16, zero_rows):
    z_out, m_out = pl.kernel(
        _edge_sc_body,
        out_type=[jax.ShapeDtypeStruct((2, NPZ, D), jnp.float32),
                  jax.ShapeDtypeStruct((2, NP, D), jnp.float32)],
        mesh=plsc.VectorSubcoreMesh(core_axis_name="c", subcore_axis_name="s"),
        scratch_types=[
            pltpu.VMEM((C,), jnp.int32),
            pltpu.VMEM((C,), jnp.int32),
            pltpu.VMEM((C,), jnp.int32),
            pltpu.VMEM((C, D), jnp.float32),
            pltpu.VMEM((C, D), jnp.float32),
            pltpu.VMEM((C, D), jnp.float32),
            pltpu.VMEM((C, D), jnp.float32),
            pltpu.VMEM((16,), jnp.float32),
            pltpu.VMEM_SHARED((NPZ, D), jnp.float32),
            pltpu.VMEM_SHARED((NP, D), jnp.float32),
            pltpu.SemaphoreType.DMA,
            pltpu.SemaphoreType.DMA,
            pltpu.SemaphoreType.DMA,
        ],
        compiler_params=pltpu.CompilerParams(needs_layout_passes=False),
    )(sp, dp, ast, h, shift16, zero_rows)
    return z_out.reshape(2, NPZ * 8, 16)[:, :NP], m_out


def _edge_phase(h, asd, shift8, sp, dp):
    el = jax.nn.leaky_relu(asd[sp, :H] + asd[dp, H:], 0.2)
    w = jnp.exp(el - shift8[None, :])
    z = jax.ops.segment_sum(w, dp, num_segments=NP)
    msg = h[sp] * jnp.repeat(w, DH, axis=1)
    ms = jax.ops.segment_sum(msg, dp, num_segments=NP)
    z16 = jnp.pad(z, ((0, 0), (0, H)))
    z2 = jnp.stack([z16, jnp.zeros_like(z16)])
    m2 = jnp.stack([ms, jnp.zeros_like(ms)])
    return z2, m2


def kernel(x, edge_index, batch, params):
    E = edge_index.shape[1]
    f32 = jnp.float32

    loop = jnp.arange(N, dtype=edge_index.dtype)
    sp = jnp.concatenate([edge_index[0], loop])
    dp = jnp.concatenate([edge_index[1], loop])
    sp = jnp.pad(sp, (0, EP - E - N), constant_values=N)
    dp = jnp.pad(dp, (0, EP - E - N), constant_values=N)

    xp = jnp.pad(x, ((0, NP - N), (0, 0)))
    zero_rows = jnp.zeros((RPT, D), f32)
    batchf = jnp.broadcast_to(
        jnp.pad(batch.astype(jnp.int32), (0, NP - N), constant_values=-1)[:, None],
        (NP, G))

    ar = jnp.arange(D)
    eye = jnp.eye(D, dtype=f32)
    ones_row = jnp.ones((1, D), f32)
    smat = (ar[None, :] // DH == jnp.arange(16)[:, None]).astype(f32)

    w1p = jnp.zeros((D, D), f32).at[:, :D // 2].set(params['W1'])
    b1p = jnp.zeros((1, D), f32).at[0, :D // 2].set(params['b1'])
    w2p = jnp.zeros((D, D), f32).at[:D // 2, :1].set(params['W2'])
    b2p = jnp.broadcast_to(params['b2'], (1, D))

    a_cur = params['Wp']
    c_cur = params['bp'][None, :]
    y = xp
    gp_row = None
    for p in params['layers']:
        am = (jnp.zeros((D, D), f32)
              .at[ar, ar // DH].set(p['a_src'].reshape(-1))
              .at[ar, H + ar // DH].set(p['a_dst'].reshape(-1)))
        h, ast = _prep(y, a_cur, c_cur, p['W'], am)
        mx = jnp.max(ast, axis=0)
        shift8 = jnp.maximum(mx[:H] + mx[H:2 * H], 0.0)
        shift16 = jnp.pad(shift8, (0, H))

        z2, m2 = _edge_sc(sp, dp, ast, h, shift16, zero_rows)

        y, ssum, ssq = _post(z2, m2, smat, p['bias'][None, :])
        mu = ssum[0] / N
        svar = _var(y, mu[None, :])
        var = svar[0] / N
        rstd = lax.rsqrt(var + 1e-5)
        gp = p['gamma'] * rstd
        gp_row = gp[None, :]
        c_cur = (p['beta'] - mu * gp)[None, :]
        a_cur = eye * gp_row

    s, c = _pool(y, gp_row, c_cur, batchf)
    out = _head(s, c, w1p, b1p, w2p, b2p)
    return out[:, :1]


# speedup vs baseline: 37.3379x; 1.1612x over previous
import functools

import jax
import jax.numpy as jnp
from jax import lax
from jax.experimental import pallas as pl
from jax.experimental.pallas import tpu as pltpu
from jax.experimental.pallas import tpu_sc as plsc

N = 10000
D = 128
H = 8
DH = 16
G = 16
BLK = 128
NP = 10112
NB = NP // BLK


def _prep_body(y_ref, a_ref, c_ref, w_ref, am_ref, h_ref, asd_ref):
    i = pl.program_id(0)
    y = y_ref[...]
    xin = jnp.dot(y, a_ref[...], preferred_element_type=jnp.float32) + c_ref[...]
    xin = jnp.maximum(xin, 0.0)
    rid = i * BLK + lax.broadcasted_iota(jnp.int32, (BLK, D), 0)
    xin = jnp.where(rid < N, xin, 0.0)
    h = jnp.dot(xin, w_ref[...], preferred_element_type=jnp.float32)
    h_ref[...] = h
    asd_ref[...] = jnp.dot(h, am_ref[...], preferred_element_type=jnp.float32)


def _prep(y, a, c, w, am):
    return pl.pallas_call(
        _prep_body,
        grid=(NB,),
        in_specs=[
            pl.BlockSpec((BLK, D), lambda i: (i, 0)),
            pl.BlockSpec((D, D), lambda i: (0, 0)),
            pl.BlockSpec((1, D), lambda i: (0, 0)),
            pl.BlockSpec((D, D), lambda i: (0, 0)),
            pl.BlockSpec((D, D), lambda i: (0, 0)),
        ],
        out_specs=[
            pl.BlockSpec((BLK, D), lambda i: (i, 0)),
            pl.BlockSpec((BLK, D), lambda i: (i, 0)),
        ],
        out_shape=[
            jax.ShapeDtypeStruct((NP, D), jnp.float32),
            jax.ShapeDtypeStruct((NP, D), jnp.float32),
        ],
    )(y, a, c, w, am)


def _post_body(z_ref, m_ref, s_ref, bias_ref, y_ref, ssum_ref, ssq_ref):
    i = pl.program_id(0)
    z = z_ref[0] + z_ref[1]
    zinv = 1.0 / (z + 1e-16)
    zb = jnp.dot(zinv, s_ref[...], preferred_element_type=jnp.float32)
    y = (m_ref[0] + m_ref[1]) * zb + bias_ref[...]
    rid = i * BLK + lax.broadcasted_iota(jnp.int32, (BLK, D), 0)
    y = jnp.where(rid < N, y, 0.0)
    y_ref[...] = y

    @pl.when(i == 0)
    def _():
        ssum_ref[...] = jnp.zeros_like(ssum_ref)
        ssq_ref[...] = jnp.zeros_like(ssq_ref)

    s = jnp.sum(y, axis=0)
    sq = jnp.sum(y * y, axis=0)
    ssum_ref[...] += jnp.broadcast_to(s, (8, D))
    ssq_ref[...] += jnp.broadcast_to(sq, (8, D))


def _post(z2, m2, smat, bias):
    return pl.pallas_call(
        _post_body,
        grid=(NB,),
        in_specs=[
            pl.BlockSpec((2, BLK, 16), lambda i: (0, i, 0)),
            pl.BlockSpec((2, BLK, D), lambda i: (0, i, 0)),
            pl.BlockSpec((16, D), lambda i: (0, 0)),
            pl.BlockSpec((1, D), lambda i: (0, 0)),
        ],
        out_specs=[
            pl.BlockSpec((BLK, D), lambda i: (i, 0)),
            pl.BlockSpec((8, D), lambda i: (0, 0)),
            pl.BlockSpec((8, D), lambda i: (0, 0)),
        ],
        out_shape=[
            jax.ShapeDtypeStruct((NP, D), jnp.float32),
            jax.ShapeDtypeStruct((8, D), jnp.float32),
            jax.ShapeDtypeStruct((8, D), jnp.float32),
        ],
    )(z2, m2, smat, bias)


def _var_body(y_ref, mu_ref, sv_ref):
    i = pl.program_id(0)
    d = y_ref[...] - mu_ref[...]
    rid = i * BLK + lax.broadcasted_iota(jnp.int32, (BLK, D), 0)
    d = jnp.where(rid < N, d, 0.0)

    @pl.when(i == 0)
    def _():
        sv_ref[...] = jnp.zeros_like(sv_ref)

    sv_ref[...] += jnp.broadcast_to(jnp.sum(d * d, axis=0), (8, D))


def _var(y, mu_row):
    return pl.pallas_call(
        _var_body,
        grid=(NB,),
        in_specs=[
            pl.BlockSpec((BLK, D), lambda i: (i, 0)),
            pl.BlockSpec((1, D), lambda i: (0, 0)),
        ],
        out_specs=pl.BlockSpec((8, D), lambda i: (0, 0)),
        out_shape=jax.ShapeDtypeStruct((8, D), jnp.float32),
    )(y, mu_row)


def _pool_body(y_ref, g_ref, b_ref, bf_ref, s_ref, c_ref):
    i = pl.program_id(0)
    x = jnp.maximum(y_ref[...] * g_ref[...] + b_ref[...], 0.0)
    rid = i * BLK + lax.broadcasted_iota(jnp.int32, (BLK, D), 0)
    x = jnp.where(rid < N, x, 0.0)
    seg = lax.broadcasted_iota(jnp.int32, (BLK, G), 1)
    oh = (bf_ref[...] == seg).astype(jnp.float32)

    @pl.when(i == 0)
    def _():
        s_ref[...] = jnp.zeros_like(s_ref)
        c_ref[...] = jnp.zeros_like(c_ref)

    s_ref[...] += lax.dot_general(oh, x, (((0,), (0,)), ((), ())),
                                  preferred_element_type=jnp.float32)
    c_ref[...] += jnp.broadcast_to(jnp.sum(oh, axis=0)[:, None], (G, D))


def _pool(y, g, b, batchf):
    return pl.pallas_call(
        _pool_body,
        grid=(NB,),
        in_specs=[
            pl.BlockSpec((BLK, D), lambda i: (i, 0)),
            pl.BlockSpec((1, D), lambda i: (0, 0)),
            pl.BlockSpec((1, D), lambda i: (0, 0)),
            pl.BlockSpec((BLK, G), lambda i: (i, 0)),
        ],
        out_specs=[
            pl.BlockSpec((G, D), lambda i: (0, 0)),
            pl.BlockSpec((G, D), lambda i: (0, 0)),
        ],
        out_shape=[
            jax.ShapeDtypeStruct((G, D), jnp.float32),
            jax.ShapeDtypeStruct((G, D), jnp.float32),
        ],
    )(y, g, b, batchf)


def _head_body(s_ref, c_ref, w1_ref, b1_ref, w2_ref, b2_ref, o_ref):
    pooled = s_ref[...] / jnp.maximum(c_ref[...], 1.0)
    hdd = jnp.maximum(
        jnp.dot(pooled, w1_ref[...], preferred_element_type=jnp.float32)
        + b1_ref[...], 0.0)
    o_ref[...] = jnp.dot(hdd, w2_ref[...],
                         preferred_element_type=jnp.float32) + b2_ref[...]


def _head(s, c, w1p, b1p, w2p, b2p):
    return pl.pallas_call(
        _head_body,
        grid=(1,),
        in_specs=[pl.BlockSpec((G, D), lambda i: (0, 0)),
                  pl.BlockSpec((G, D), lambda i: (0, 0)),
                  pl.BlockSpec((D, D), lambda i: (0, 0)),
                  pl.BlockSpec((1, D), lambda i: (0, 0)),
                  pl.BlockSpec((D, D), lambda i: (0, 0)),
                  pl.BlockSpec((1, D), lambda i: (0, 0))],
        out_specs=pl.BlockSpec((G, D), lambda i: (0, 0)),
        out_shape=jax.ShapeDtypeStruct((G, D), jnp.float32),
    )(s, c, w1p, b1p, w2p, b2p)


NC = 2
NSC = 16
NW = NC * NSC
C = 32
CH = -(-(320000 + N) // (C * NW))
CH += CH % 2
EP = CH * C * NW
TPT = EP // NW
RPT = NP // NSC
NPB = NP // 8
NPZ = 1280
ZRT = NPZ // NSC


def _edge_sc_body(sp_hbm, dp_hbm, ast_hbm, h_hbm, shift_hbm, zero_hbm,
                  z_out, m_out,
                  si0, di0, si1, di1, ib,
                  ra0, rb0, hb0, ra1, rb1, hb1, wb, shiftv,
                  z_sh, o_sh, semi0, semi1, semg0, semg1):
    cid = lax.axis_index("c")
    sid = lax.axis_index("s")
    wid = sid * NC + cid

    r0 = sid * RPT
    z0 = sid * ZRT
    pltpu.sync_copy(zero_hbm, o_sh.at[pl.ds(r0, RPT)])
    pltpu.sync_copy(zero_hbm.at[pl.ds(0, ZRT)], z_sh.at[pl.ds(z0, ZRT)])
    pltpu.sync_copy(shift_hbm, shiftv)
    plsc.subcore_barrier()

    shv = shiftv[...]
    zeros16 = jnp.zeros((16,), jnp.float32)
    lane = lax.iota(jnp.int32, 16)
    lane7 = lane & 7
    perm_hi = lane7 + 8
    lane_lo = lane < 8

    def fire_idx(g, sib, dib, semi):
        base = jnp.where(g < CH, wid * TPT + g * C, wid * TPT)
        pltpu.async_copy(sp_hbm.at[pl.ds(base, C)], sib, semi)
        pltpu.async_copy(dp_hbm.at[pl.ds(base, C)], dib, semi)

    def wait_idx(sib, dib, semi):
        pltpu.make_async_copy(sp_hbm.at[pl.ds(0, C)], sib, semi).wait()
        pltpu.make_async_copy(sp_hbm.at[pl.ds(0, C)], dib, semi).wait()

    def fire_gathers(sib, dib, rab, rbb, hbb, semg):
        pltpu.async_copy(ast_hbm.at[sib], rab, semg)
        pltpu.async_copy(ast_hbm.at[dib], rbb, semg)
        pltpu.async_copy(h_hbm.at[sib], hbb, semg)

    def drain_gathers(sib, rab, rbb, hbb, semg):
        pltpu.make_async_copy(ast_hbm.at[sib], rab, semg).wait()
        pltpu.make_async_copy(ast_hbm.at[sib], rbb, semg).wait()
        pltpu.make_async_copy(h_hbm.at[sib], hbb, semg).wait()

    def compute(dib, rab, rbb, hbb):
        def mkib(k, c2):
            sl = pl.ds(k * 16, 16)
            ib[sl] = lax.shift_right_logical(dib[sl], 3)
            return c2

        lax.fori_loop(0, C // 16, mkib, 0)

        def grp(j, c2):
            for r2 in range(16):
                r = j * 16 + r2
                rsp = lane * 0 + r
                e = rab[r, pl.ds(0, 16)] + plsc.load_gather(rbb, [rsp, perm_hi])
                el = jnp.where(e >= 0.0, e, 0.2 * e)
                w = jnp.where(lane_lo, jnp.exp(el - shv), 0.0)
                dspl = plsc.load_gather(dib, [rsp])
                osp = (dspl & 7) * 16
                for s in range(8):
                    wb[r, pl.ds(s * 16, 16)] = zeros16
                plsc.store_scatter(wb, [rsp, osp + lane], w)
                for hh in range(H):
                    spl = plsc.load_gather(wb, [rsp, osp + hh])
                    sl = pl.ds(hh * 16, 16)
                    hbb[r, sl] = hbb[r, sl] * spl
            return c2

        lax.fori_loop(0, C // 16, grp, 0)
        pltpu.sync_copy(wb, z_sh.at[ib], add=True)
        pltpu.sync_copy(hbb, o_sh.at[dib], add=True)

    fire_idx(0, si0, di0, semi0)
    wait_idx(si0, di0, semi0)
    fire_gathers(si0, di0, ra0, rb0, hb0, semg0)
    fire_idx(1, si1, di1, semi1)

    def pipe(k, carry):
        g0 = 2 * k
        wait_idx(si1, di1, semi1)
        fire_gathers(si1, di1, ra1, rb1, hb1, semg1)
        drain_gathers(si0, ra0, rb0, hb0, semg0)
        compute(di0, ra0, rb0, hb0)
        fire_idx(g0 + 2, si0, di0, semi0)
        wait_idx(si0, di0, semi0)
        fire_gathers(si0, di0, ra0, rb0, hb0, semg0)
        drain_gathers(si1, ra1, rb1, hb1, semg1)
        compute(di1, ra1, rb1, hb1)
        fire_idx(g0 + 3, si1, di1, semi1)
        return carry

    lax.fori_loop(0, CH // 2, pipe, 0)
    drain_gathers(si0, ra0, rb0, hb0, semg0)
    wait_idx(si1, di1, semi1)
    plsc.subcore_barrier()
    pltpu.sync_copy(z_sh.at[pl.ds(z0, ZRT)], z_out.at[cid, pl.ds(z0, ZRT)])
    pltpu.sync_copy(o_sh.at[pl.ds(r0, RPT)], m_out.at[cid, pl.ds(r0, RPT)])


def _edge_sc(sp, dp, ast, h, shift16, zero_rows):
    z_out, m_out = pl.kernel(
        _edge_sc_body,
        out_type=[jax.ShapeDtypeStruct((2, NPZ, D), jnp.float32),
                  jax.ShapeDtypeStruct((2, NP, D), jnp.float32)],
        mesh=plsc.VectorSubcoreMesh(core_axis_name="c", subcore_axis_name="s"),
        scratch_types=[
            pltpu.VMEM((C,), jnp.int32),
            pltpu.VMEM((C,), jnp.int32),
            pltpu.VMEM((C,), jnp.int32),
            pltpu.VMEM((C,), jnp.int32),
            pltpu.VMEM((C,), jnp.int32),
            pltpu.VMEM((C, D), jnp.float32),
            pltpu.VMEM((C, D), jnp.float32),
            pltpu.VMEM((C, D), jnp.float32),
            pltpu.VMEM((C, D), jnp.float32),
            pltpu.VMEM((C, D), jnp.float32),
            pltpu.VMEM((C, D), jnp.float32),
            pltpu.VMEM((C, D), jnp.float32),
            pltpu.VMEM((16,), jnp.float32),
            pltpu.VMEM_SHARED((NPZ, D), jnp.float32),
            pltpu.VMEM_SHARED((NP, D), jnp.float32),
            pltpu.SemaphoreType.DMA,
            pltpu.SemaphoreType.DMA,
            pltpu.SemaphoreType.DMA,
            pltpu.SemaphoreType.DMA,
        ],
        compiler_params=pltpu.CompilerParams(needs_layout_passes=False),
    )(sp, dp, ast, h, shift16, zero_rows)
    return z_out.reshape(2, NPZ * 8, 16)[:, :NP], m_out


def _edge_phase(h, asd, shift8, sp, dp):
    el = jax.nn.leaky_relu(asd[sp, :H] + asd[dp, H:], 0.2)
    w = jnp.exp(el - shift8[None, :])
    z = jax.ops.segment_sum(w, dp, num_segments=NP)
    msg = h[sp] * jnp.repeat(w, DH, axis=1)
    ms = jax.ops.segment_sum(msg, dp, num_segments=NP)
    z16 = jnp.pad(z, ((0, 0), (0, H)))
    z2 = jnp.stack([z16, jnp.zeros_like(z16)])
    m2 = jnp.stack([ms, jnp.zeros_like(ms)])
    return z2, m2


def kernel(x, edge_index, batch, params):
    E = edge_index.shape[1]
    f32 = jnp.float32

    loop = jnp.arange(N, dtype=edge_index.dtype)
    sp = jnp.concatenate([edge_index[0], loop])
    dp = jnp.concatenate([edge_index[1], loop])
    sp = jnp.pad(sp, (0, EP - E - N), constant_values=N)
    dp = jnp.pad(dp, (0, EP - E - N), constant_values=N)

    xp = jnp.pad(x, ((0, NP - N), (0, 0)))
    zero_rows = jnp.zeros((RPT, D), f32)
    batchf = jnp.broadcast_to(
        jnp.pad(batch.astype(jnp.int32), (0, NP - N), constant_values=-1)[:, None],
        (NP, G))

    ar = jnp.arange(D)
    eye = jnp.eye(D, dtype=f32)
    ones_row = jnp.ones((1, D), f32)
    smat = (ar[None, :] // DH == jnp.arange(16)[:, None]).astype(f32)

    w1p = jnp.zeros((D, D), f32).at[:, :D // 2].set(params['W1'])
    b1p = jnp.zeros((1, D), f32).at[0, :D // 2].set(params['b1'])
    w2p = jnp.zeros((D, D), f32).at[:D // 2, :1].set(params['W2'])
    b2p = jnp.broadcast_to(params['b2'], (1, D))

    a_cur = params['Wp']
    c_cur = params['bp'][None, :]
    y = xp
    gp_row = None
    for p in params['layers']:
        am = (jnp.zeros((D, D), f32)
              .at[ar, ar // DH].set(p['a_src'].reshape(-1))
              .at[ar, H + ar // DH].set(p['a_dst'].reshape(-1)))
        h, ast = _prep(y, a_cur, c_cur, p['W'], am)
        mx = jnp.max(ast, axis=0)
        shift8 = jnp.maximum(mx[:H] + mx[H:2 * H], 0.0)
        shift16 = jnp.pad(shift8, (0, H))

        z2, m2 = _edge_sc(sp, dp, ast, h, shift16, zero_rows)

        y, ssum, ssq = _post(z2, m2, smat, p['bias'][None, :])
        mu = ssum[0] / N
        svar = _var(y, mu[None, :])
        var = svar[0] / N
        rstd = lax.rsqrt(var + 1e-5)
        gp = p['gamma'] * rstd
        gp_row = gp[None, :]
        c_cur = (p['beta'] - mu * gp)[None, :]
        a_cur = eye * gp_row

    s, c = _pool(y, gp_row, c_cur, batchf)
    out = _head(s, c, w1p, b1p, w2p, b2p)
    return out[:, :1]
